# baseline (speedup 1.0000x reference)
import functools

import jax
import jax.numpy as jnp
from jax import lax
from jax.experimental import pallas as pl
from jax.experimental.pallas import tpu as pltpu
from jax.experimental.pallas import tpu_sc as plsc

_NUM_BITS = 16
_SCALE = 0.01
_QMAX = 2 ** _NUM_BITS - 1
_ZP = 1 << (_NUM_BITS - 1)

_LANES = 16
_NW = 32
_ROWS = 8
_CSPLIT = 2

_RND = 12582912.0
_RND_BITS = 0x4B400000


def _sc_body(x_hbm, t01_hbm, out_hbm, t01_v, xb0, xb1, ob0, ob1,
             si0, si1, so0, so1):
    rows, cols = x_hbm.shape
    ccols = cols // _CSPLIT
    rows_w = rows // _NW
    n_chunks = rows_w // _ROWS * _CSPLIT

    wid = lax.axis_index("s") * 2 + lax.axis_index("c")
    base = wid * rows_w

    pltpu.sync_copy(t01_hbm, t01_v)

    xbs, obs, sis, sos = (xb0, xb1), (ob0, ob1), (si0, si1), (so0, so1)

    def hbm_slice(c):
        r0 = base + (c // _CSPLIT) * _ROWS
        c0 = (c % _CSPLIT) * ccols
        return (pl.ds(r0, _ROWS), pl.ds(c0, ccols))

    def in_copy(c, b):
        return pltpu.make_async_copy(x_hbm.at[hbm_slice(c)], xbs[b], sis[b])

    def out_copy(c, b):
        return pltpu.make_async_copy(obs[b], out_hbm.at[hbm_slice(c)], sos[b])

    def compute(b):
        xb, ob = xbs[b], obs[b]
        for r in range(_ROWS):
            @plsc.parallel_loop(0, ccols, _LANES, unroll=8)
            def _(i):
                xv = xb[r, pl.ds(i, _LANES)]
                v = xv * jnp.float32(1.0 / _SCALE) + jnp.float32(_RND + _ZP)
                q = plsc.bitcast(v, jnp.int32) - _RND_BITS
                q = jnp.minimum(jnp.maximum(q, 0), _QMAX)
                ob[r, pl.ds(i, _LANES)] = plsc.load_gather(t01_v, [q])

    in_copy(0, 0).start()
    in_copy(1, 1).start()

    def step(c, b):
        in_copy(c, b).wait()

        @pl.when(c >= 2)
        def _():
            out_copy(c - 2, b).wait()

        compute(b)
        out_copy(c, b).start()

        @pl.when(c + 2 < n_chunks)
        def _():
            in_copy(c + 2, b).start()

    def body(g, carry):
        c0 = g * 2
        for b in range(2):
            step(c0 + b, b)
        return carry

    lax.fori_loop(0, n_chunks // 2, body, 0)
    out_copy(n_chunks - 2, 0).wait()
    out_copy(n_chunks - 1, 1).wait()


def kernel(x, t0, t1):
    shape = x.shape
    x2 = x.reshape(-1, shape[-1])
    rows, cols = x2.shape
    ccols = cols // _CSPLIT
    assert rows % (_NW * _ROWS) == 0 and ccols % 128 == 0

    t01 = (t1[:, None] + t0[None, :]).reshape(-1)

    mesh = plsc.VectorSubcoreMesh(core_axis_name="c", subcore_axis_name="s")
    f = functools.partial(
        pl.kernel,
        out_type=jax.ShapeDtypeStruct((rows, cols), jnp.float32),
        mesh=mesh,
        compiler_params=pltpu.CompilerParams(
            needs_layout_passes=False, use_tc_tiling_on_sc=True),
        scratch_types=[
            pltpu.VMEM((_QMAX + 1,), jnp.float32),
            pltpu.VMEM((_ROWS, ccols), jnp.float32),
            pltpu.VMEM((_ROWS, ccols), jnp.float32),
            pltpu.VMEM((_ROWS, ccols), jnp.float32),
            pltpu.VMEM((_ROWS, ccols), jnp.float32),
            pltpu.SemaphoreType.DMA,
            pltpu.SemaphoreType.DMA,
            pltpu.SemaphoreType.DMA,
            pltpu.SemaphoreType.DMA,
        ],
    )(_sc_body)
    out = f(x2, t01)
    return out.reshape(shape)


# speedup vs baseline: 1.1272x; 1.1272x over previous
import functools

import jax
import jax.numpy as jnp
from jax import lax
from jax.experimental import pallas as pl
from jax.experimental.pallas import tpu as pltpu
from jax.experimental.pallas import tpu_sc as plsc

_NUM_BITS = 16
_SCALE = 0.01
_QMAX = 2 ** _NUM_BITS - 1
_ZP = 1 << (_NUM_BITS - 1)

_LANES = 16
_NW = 32
_ROWS = 8
_NBUF = 3

_RND = 12582912.0
_RND_BITS = 0x4B400000


def _sc_body(x_hbm, t01_hbm, out_hbm, t01_v, b0, b1, b2,
             si0, si1, si2, so0, so1, so2):
    rows, cols = x_hbm.shape
    rows_w = rows // _NW
    n_chunks = rows_w // _ROWS

    wid = lax.axis_index("s") * 2 + lax.axis_index("c")
    base = wid * rows_w

    pltpu.sync_copy(t01_hbm, t01_v)

    bufs, sis, sos = (b0, b1, b2), (si0, si1, si2), (so0, so1, so2)

    def in_copy(c, b):
        return pltpu.make_async_copy(
            x_hbm.at[pl.ds(base + c * _ROWS, _ROWS), :], bufs[b], sis[b])

    def out_copy(c, b):
        return pltpu.make_async_copy(
            bufs[b], out_hbm.at[pl.ds(base + c * _ROWS, _ROWS), :], sos[b])

    def compute(b):
        buf = bufs[b]
        for r in range(_ROWS):
            @plsc.parallel_loop(0, cols, _LANES, unroll=8)
            def _(i):
                xv = buf[r, pl.ds(i, _LANES)]
                v = xv * jnp.float32(1.0 / _SCALE) + jnp.float32(_RND + _ZP)
                q = plsc.bitcast(v, jnp.int32) - _RND_BITS
                q = jnp.minimum(jnp.maximum(q, 0), _QMAX)
                buf[r, pl.ds(i, _LANES)] = plsc.load_gather(t01_v, [q])

    in_copy(0, 0).start()
    in_copy(1, 1).start()

    def step(c, b):
        in_copy(c, b).wait()
        compute(b)
        out_copy(c, b).start()

        prev = (b + _NBUF - 1) % _NBUF

        @pl.when(c >= 1)
        def _():
            out_copy(c - 1, prev).wait()

        @pl.when(c + 2 < n_chunks)
        def _():
            in_copy(c + 2, prev).start()

    def body(g, carry):
        c0 = g * _NBUF
        for b in range(_NBUF):
            step(c0 + b, b)
        return carry

    n_main = n_chunks // _NBUF * _NBUF
    lax.fori_loop(0, n_chunks // _NBUF, body, 0)
    for cc in range(n_main, n_chunks):
        step(cc, cc % _NBUF)

    out_copy(n_chunks - 1, (n_chunks - 1) % _NBUF).wait()


def kernel(x, t0, t1):
    shape = x.shape
    x2 = x.reshape(-1, shape[-1])
    rows, cols = x2.shape
    assert rows % (_NW * _ROWS) == 0 and cols % _LANES == 0

    t01 = (t1[:, None] + t0[None, :]).reshape(-1)

    mesh = plsc.VectorSubcoreMesh(core_axis_name="c", subcore_axis_name="s")
    f = functools.partial(
        pl.kernel,
        out_type=jax.ShapeDtypeStruct((rows, cols), jnp.float32),
        mesh=mesh,
        compiler_params=pltpu.CompilerParams(
            needs_layout_passes=False, use_tc_tiling_on_sc=True),
        scratch_types=[
            pltpu.VMEM((_QMAX + 1,), jnp.float32),
            pltpu.VMEM((_ROWS, cols), jnp.float32),
            pltpu.VMEM((_ROWS, cols), jnp.float32),
            pltpu.VMEM((_ROWS, cols), jnp.float32),
            pltpu.SemaphoreType.DMA,
            pltpu.SemaphoreType.DMA,
            pltpu.SemaphoreType.DMA,
            pltpu.SemaphoreType.DMA,
            pltpu.SemaphoreType.DMA,
            pltpu.SemaphoreType.DMA,
        ],
    )(_sc_body)
    out = f(x2, t01)
    return out.reshape(shape)


# speedup vs baseline: 1.1681x; 1.0363x over previous
import functools

import jax
import jax.numpy as jnp
from jax import lax
from jax.experimental import pallas as pl
from jax.experimental.pallas import tpu as pltpu
from jax.experimental.pallas import tpu_sc as plsc

_NUM_BITS = 16
_SCALE = 0.01
_QMAX = 2 ** _NUM_BITS - 1
_ZP = 1 << (_NUM_BITS - 1)

_LANES = 16
_NW = 32
_ROWS = 8
_NBUF = 3

_RND = 12582912.0
_RND_BITS = 0x4B400000


def _sc_body(x_hbm, t01_hbm, out_hbm, t01_v, b0, b1, b2,
             si0, si1, si2, so0, so1, so2):
    rows, cols = x_hbm.shape
    rows_w = rows // _NW
    n_chunks = rows_w // _ROWS

    wid = lax.axis_index("s") * 2 + lax.axis_index("c")
    base = wid * rows_w

    pltpu.sync_copy(t01_hbm, t01_v)

    bufs, sis, sos = (b0, b1, b2), (si0, si1, si2), (so0, so1, so2)

    def in_copy(c, b):
        return pltpu.make_async_copy(
            x_hbm.at[pl.ds(base + c * _ROWS, _ROWS), :], bufs[b], sis[b])

    def out_copy(c, b):
        return pltpu.make_async_copy(
            bufs[b], out_hbm.at[pl.ds(base + c * _ROWS, _ROWS), :], sos[b])

    def compute(b):
        buf = bufs[b]
        if True:
            @plsc.parallel_loop(0, cols, _LANES, unroll=2)
            def _(i):
              for r in range(_ROWS):
                xv = buf[r, pl.ds(i, _LANES)]
                v = xv * jnp.float32(1.0 / _SCALE) + jnp.float32(_RND + _ZP)
                q = plsc.bitcast(v, jnp.int32) - _RND_BITS
                q = jnp.minimum(jnp.maximum(q, 0), _QMAX)
                buf[r, pl.ds(i, _LANES)] = plsc.load_gather(t01_v, [q])

    in_copy(0, 0).start()
    in_copy(1, 1).start()

    def step(c, b):
        in_copy(c, b).wait()
        compute(b)
        out_copy(c, b).start()

        prev = (b + _NBUF - 1) % _NBUF

        @pl.when(c >= 1)
        def _():
            out_copy(c - 1, prev).wait()

        @pl.when(c + 2 < n_chunks)
        def _():
            in_copy(c + 2, prev).start()

    def body(g, carry):
        c0 = g * _NBUF
        for b in range(_NBUF):
            step(c0 + b, b)
        return carry

    n_main = n_chunks // _NBUF * _NBUF
    lax.fori_loop(0, n_chunks // _NBUF, body, 0)
    for cc in range(n_main, n_chunks):
        step(cc, cc % _NBUF)

    out_copy(n_chunks - 1, (n_chunks - 1) % _NBUF).wait()


def kernel(x, t0, t1):
    shape = x.shape
    x2 = x.reshape(-1, shape[-1])
    rows, cols = x2.shape
    assert rows % (_NW * _ROWS) == 0 and cols % _LANES == 0

    t01 = (t1[:, None] + t0[None, :]).reshape(-1)

    mesh = plsc.VectorSubcoreMesh(core_axis_name="c", subcore_axis_name="s")
    f = functools.partial(
        pl.kernel,
        out_type=jax.ShapeDtypeStruct((rows, cols), jnp.float32),
        mesh=mesh,
        compiler_params=pltpu.CompilerParams(
            needs_layout_passes=False, use_tc_tiling_on_sc=True),
        scratch_types=[
            pltpu.VMEM((_QMAX + 1,), jnp.float32),
            pltpu.VMEM((_ROWS, cols), jnp.float32),
            pltpu.VMEM((_ROWS, cols), jnp.float32),
            pltpu.VMEM((_ROWS, cols), jnp.float32),
            pltpu.SemaphoreType.DMA,
            pltpu.SemaphoreType.DMA,
            pltpu.SemaphoreType.DMA,
            pltpu.SemaphoreType.DMA,
            pltpu.SemaphoreType.DMA,
            pltpu.SemaphoreType.DMA,
        ],
    )(_sc_body)
    out = f(x2, t01)
    return out.reshape(shape)


# speedup vs baseline: 1.1864x; 1.0156x over previous
import functools

import jax
import jax.numpy as jnp
from jax import lax
from jax.experimental import pallas as pl
from jax.experimental.pallas import tpu as pltpu
from jax.experimental.pallas import tpu_sc as plsc

_NUM_BITS = 16
_SCALE = 0.01
_QMAX = 2 ** _NUM_BITS - 1
_ZP = 1 << (_NUM_BITS - 1)

_LANES = 16
_NW = 32
_ROWS = 8
_NBUF = 3

_RND = 12582912.0
_RND_BITS = 0x4B400000


def _sc_body(x_hbm, t01_hbm, out_hbm, t01_v, b0, b1, b2,
             si0, si1, si2, so0, so1, so2):
    rows, cols = x_hbm.shape
    rows_w = rows // _NW
    n_chunks = rows_w // _ROWS

    wid = lax.axis_index("s") * 2 + lax.axis_index("c")
    base = wid * rows_w

    pltpu.sync_copy(t01_hbm, t01_v)

    bufs, sis, sos = (b0, b1, b2), (si0, si1, si2), (so0, so1, so2)

    def in_copy(c, b):
        return pltpu.make_async_copy(
            x_hbm.at[pl.ds(base + c * _ROWS, _ROWS), :], bufs[b], sis[b])

    def out_copy(c, b):
        return pltpu.make_async_copy(
            bufs[b], out_hbm.at[pl.ds(base + c * _ROWS, _ROWS), :], sos[b])

    def compute(b):
        buf = bufs[b]
        if True:
            @plsc.parallel_loop(0, cols, _LANES, unroll=1)
            def _(i):
              for r in range(_ROWS):
                xv = buf[r, pl.ds(i, _LANES)]
                v = xv * jnp.float32(1.0 / _SCALE) + jnp.float32(_RND + _ZP)
                q = plsc.bitcast(v, jnp.int32) - _RND_BITS
                q = jnp.minimum(jnp.maximum(q, 0), _QMAX)
                buf[r, pl.ds(i, _LANES)] = plsc.load_gather(t01_v, [q])

    in_copy(0, 0).start()
    in_copy(1, 1).start()

    def step(c, b):
        in_copy(c, b).wait()
        compute(b)
        out_copy(c, b).start()

        prev = (b + _NBUF - 1) % _NBUF

        @pl.when(c >= 1)
        def _():
            out_copy(c - 1, prev).wait()

        @pl.when(c + 2 < n_chunks)
        def _():
            in_copy(c + 2, prev).start()

    def body(g, carry):
        c0 = g * _NBUF
        for b in range(_NBUF):
            step(c0 + b, b)
        return carry

    n_main = n_chunks // _NBUF * _NBUF
    lax.fori_loop(0, n_chunks // _NBUF, body, 0)
    for cc in range(n_main, n_chunks):
        step(cc, cc % _NBUF)

    out_copy(n_chunks - 1, (n_chunks - 1) % _NBUF).wait()


def kernel(x, t0, t1):
    shape = x.shape
    x2 = x.reshape(-1, shape[-1])
    rows, cols = x2.shape
    assert rows % (_NW * _ROWS) == 0 and cols % _LANES == 0

    t01 = (t1[:, None] + t0[None, :]).reshape(-1)

    mesh = plsc.VectorSubcoreMesh(core_axis_name="c", subcore_axis_name="s")
    f = functools.partial(
        pl.kernel,
        out_type=jax.ShapeDtypeStruct((rows, cols), jnp.float32),
        mesh=mesh,
        compiler_params=pltpu.CompilerParams(
            needs_layout_passes=False, use_tc_tiling_on_sc=True),
        scratch_types=[
            pltpu.VMEM((_QMAX + 1,), jnp.float32),
            pltpu.VMEM((_ROWS, cols), jnp.float32),
            pltpu.VMEM((_ROWS, cols), jnp.float32),
            pltpu.VMEM((_ROWS, cols), jnp.float32),
            pltpu.SemaphoreType.DMA,
            pltpu.SemaphoreType.DMA,
            pltpu.SemaphoreType.DMA,
            pltpu.SemaphoreType.DMA,
            pltpu.SemaphoreType.DMA,
            pltpu.SemaphoreType.DMA,
        ],
    )(_sc_body)
    out = f(x2, t01)
    return out.reshape(shape)
